# baseline (speedup 1.0000x reference)
import functools

import jax
import jax.numpy as jnp
from jax.experimental import pallas as pl
from jax.experimental.pallas import tpu as pltpu

N = 1000000
D = 16
_SIM_BLOCK = 8000


def _sim_body(q_ref, lat_ref, out_ref):
    p = lat_ref[...] * q_ref[...]
    t = p[:, :8] + p[:, 8:]
    t = t[:, :4] + t[:, 4:]
    t = t[:, :2] + t[:, 2:]
    out_ref[...] = (t[:, 0] + t[:, 1]).reshape(1, 1, _SIM_BLOCK)


def _similarities(query_latent, latents):
    n = latents.shape[0]
    nblk = n // _SIM_BLOCK
    with jax.enable_x64(False):
        out = pl.pallas_call(
            _sim_body,
            grid=(nblk,),
            in_specs=[
                pl.BlockSpec((1, D), lambda i: (0, 0)),
                pl.BlockSpec((_SIM_BLOCK, D), lambda i: (i, 0)),
            ],
            out_specs=pl.BlockSpec((1, 1, _SIM_BLOCK), lambda i: (i, 0, 0)),
            out_shape=jax.ShapeDtypeStruct((nblk, 1, _SIM_BLOCK), jnp.float32),
        )(query_latent.reshape(1, D), latents)
    return out.reshape(n)


def kernel(query_latent, latents, latent_metadatas, max_results=50):
    sims = _similarities(query_latent, latents)
    order = jnp.argsort(-sims)
    sims_sorted = sims[order]
    relevant_metadata = latent_metadatas[order]
    return relevant_metadata, sims_sorted, latents
